# baseline (speedup 1.0000x reference)
import functools

import jax
import jax.numpy as jnp
import numpy as np
from jax import lax
from jax.experimental import pallas as pl
from jax.experimental.pallas import tpu as pltpu
from jax.experimental.pallas import tpu_sc as plsc

DIM = 768
HEADS = 12
DIM_HEAD = 64
SEQ = 4096
BUCKET_SIZE = 64
N_HASHES = 4
N_BUCKETS = SEQ // BUCKET_SIZE
N_CHUNKS = N_HASHES * N_BUCKETS
TOKEN_SELF_ATTN_VALUE = -5e4

_F32 = jnp.float32


def _tri_consts():
    i = np.arange(64)
    t_incl = (i[None, :] <= i[:, None]).astype(np.float32)
    t_excl = (i[None, :] < i[:, None]).astype(np.float32)
    u_strict = (i[:, None] < i[None, :]).astype(np.float32)
    return jnp.asarray(t_incl), jnp.asarray(t_excl), jnp.asarray(u_strict)


def _diag_mask():
    m = np.zeros((64, 128), np.float32)
    m[np.arange(64), np.arange(64)] = 1.0
    return jnp.asarray(m)


def _proj_hash_body(x_ref, wqk_ref, wv_ref, rot_ref, ti_ref, te_ref, us_ref,
                    comb_ref, gpos_ref, btid_ref):
    h = pl.program_id(0)
    x = x_ref[...]
    qk = jnp.dot(x, wqk_ref[0], preferred_element_type=_F32)
    v = jnp.dot(x, wv_ref[0], preferred_element_type=_F32)
    comb_ref[0, :, 0:DIM_HEAD] = qk
    comb_ref[0, :, DIM_HEAD:2 * DIM_HEAD] = v

    p = jnp.dot(qk, rot_ref[...], preferred_element_type=_F32)
    t_incl, t_excl, u_strict = ti_ref[...], te_ref[...], us_ref[...]
    lane = jax.lax.broadcasted_iota(jnp.int32, (SEQ, N_BUCKETS), 1)
    for r in range(N_HASHES):
        pr = p[:, r * 32:(r + 1) * 32]
        scores = jnp.concatenate([pr, -pr], axis=1)
        m = jnp.max(scores, axis=1, keepdims=True)
        b_idx = jnp.min(jnp.where(scores == m, lane, N_BUCKETS),
                        axis=1, keepdims=True)
        onehot = (lane == b_idx).astype(_F32)
        o3 = onehot.reshape(64, 64, N_BUCKETS)
        inner = jnp.einsum('ij,bjk->bik', t_incl, o3,
                           preferred_element_type=_F32)
        s_blk = jnp.sum(o3, axis=1)
        blk_offs = jnp.dot(t_excl, s_blk, preferred_element_type=_F32)
        counts = jnp.sum(s_blk, axis=0, keepdims=True)
        b_offs = jnp.dot(counts, u_strict, preferred_element_type=_F32)
        posmat = inner - 1.0 + blk_offs[:, None, :] + b_offs[None, :, :]
        pos = jnp.sum(o3 * posmat, axis=2)
        base = (h * N_HASHES + r) * SEQ
        gpos = pos.astype(jnp.int32) + base
        gpos_ref[0, r * 4:(r + 1) * 4, :] = gpos.reshape(4, 1024)

        pos_flat = pos.reshape(SEQ, 1)
        slot_lane = jax.lax.broadcasted_iota(jnp.int32, (SEQ, 128), 1)
        slots = jnp.where(slot_lane < 64, slot_lane, slot_lane + (SEQ - 128))
        hit = (pos_flat.astype(jnp.int32) == slots).astype(_F32)
        tvec = jax.lax.broadcasted_iota(jnp.int32, (SEQ, 128), 0).astype(_F32)
        btid = jnp.sum(tvec * hit, axis=0, keepdims=True)
        btid_ref[0, 2 * r:2 * r + 1, :] = btid[:, 0:64]
        btid_ref[0, 2 * r + 1:2 * r + 2, :] = btid[:, 64:128]


def _attn_body(cur_ref, halo_ref, btid_ref, dmask_ref, out_ref):
    q = pl.program_id(1)
    cur = cur_ref[0]
    halo = halo_ref[0, 0]
    prevs = jnp.concatenate([halo[None], cur[:-1]], axis=0)
    kv = jnp.concatenate([cur, prevs], axis=1)
    bq = cur[:, :, 0:DIM_HEAD]
    bqk_k = kv[:, :, 0:DIM_HEAD]
    norm = jnp.sqrt(jnp.sum(bqk_k * bqk_k, axis=2, keepdims=True))
    bk = bqk_k / jnp.maximum(norm, 1e-12)
    bv = kv[:, :, DIM_HEAD:2 * DIM_HEAD]
    dots = jax.lax.dot_general(
        bq, bk, (((2,), (2,)), ((0,), (0,))),
        preferred_element_type=_F32) * (DIM_HEAD ** -0.5)
    dm = dmask_ref[...][None]
    dots = dots - dm * dots + TOKEN_SELF_ATTN_VALUE * dm
    qt_b = btid_ref[0, pl.ds(2 * q, 1), :]
    r_prev = (q + N_HASHES - 1) % N_HASHES
    kt_b = btid_ref[0, pl.ds(2 * r_prev + 1, 1), :]
    ktx = jnp.concatenate(
        [jnp.full((1, 64), -1.0, _F32), kt_b], axis=1)
    qt_col = jnp.broadcast_to(qt_b, (64, 64)).T[:, 0:1]
    bnd = (qt_col[None] == ktx[:, None, :]).astype(_F32)
    d0 = dots[0:1]
    d0 = d0 - bnd * d0 + TOKEN_SELF_ATTN_VALUE * bnd
    dots = jnp.concatenate([d0, dots[1:]], axis=0)
    m = jnp.max(dots, axis=2, keepdims=True)
    ex = jnp.exp(dots - m)
    s = jnp.sum(ex, axis=2, keepdims=True)
    lse = m + jnp.log(s)
    pr = ex / s
    bo = jax.lax.dot_general(
        pr, bv, (((2,), (1,)), ((0,), (0,))),
        preferred_element_type=_F32)
    out_ref[0, :, :, 0:DIM_HEAD] = bo
    out_ref[0, :, :, DIM_HEAD:2 * DIM_HEAD] = jnp.broadcast_to(
        lse, (64, 64, DIM_HEAD))


def _combine_body(u0_ref, u1_ref, wout_ref, out_ref):
    ys = []
    for ref in (u0_ref, u1_ref):
        u = ref[...]
        logit = u[:, :, :, DIM_HEAD:DIM_HEAD + 1]
        m = jnp.max(logit, axis=1, keepdims=True)
        ex = jnp.exp(logit - m)
        s = jnp.sum(ex, axis=1, keepdims=True)
        lse = m + jnp.log(s)
        probs = jnp.exp(logit - lse)
        y_h = jnp.sum(u[:, :, :, 0:DIM_HEAD] * probs, axis=1)
        ys.extend([y_h[i] for i in range(y_h.shape[0])])
    y = jnp.concatenate(ys, axis=1)
    out_ref[...] = jnp.dot(y, wout_ref[...], preferred_element_type=_F32)


SC_WIN = 128
SC_NW = 32

def _sc_mesh():
    return plsc.VectorSubcoreMesh(core_axis_name="c", subcore_axis_name="s")


def _sc_scatter_rows(comb2, gpos):
    d = comb2.shape[1]
    n = gpos.shape[0]
    wins_per_w = n // (SC_WIN * SC_NW)

    @functools.partial(
        pl.kernel, mesh=_sc_mesh(),
        out_type=jax.ShapeDtypeStruct((n, d), comb2.dtype),
        scratch_types=[
            pltpu.VMEM((SC_WIN,), jnp.int32),
            pltpu.VMEM((SC_WIN, d), comb2.dtype),
        ],
    )
    def _k(comb_hbm, idx_hbm, out_hbm, idx_v, rows_v):
        wid = lax.axis_index("s") * 2 + lax.axis_index("c")

        @pl.loop(0, wins_per_w)
        def _(w):
            base = (wid * wins_per_w + w) * SC_WIN
            src = (base // (N_HASHES * SEQ)) * SEQ + base % SEQ
            pltpu.sync_copy(idx_hbm.at[pl.ds(base, SC_WIN)], idx_v)
            pltpu.sync_copy(comb_hbm.at[pl.ds(src, SC_WIN)], rows_v)
            pltpu.sync_copy(rows_v, out_hbm.at[idx_v])

    return _k(comb2, gpos)


def _sc_gather_rows(table, gpos):
    d = table.shape[1]
    n = gpos.shape[0]
    wins_per_w = n // (SC_WIN * SC_NW)

    @functools.partial(
        pl.kernel, mesh=_sc_mesh(),
        out_type=jax.ShapeDtypeStruct((n, d), table.dtype),
        scratch_types=[
            pltpu.VMEM((SC_WIN,), jnp.int32),
            pltpu.VMEM((SC_WIN, d), table.dtype),
            pltpu.SemaphoreType.DMA,
        ],
    )
    def _k(table_hbm, idx_hbm, out_hbm, idx_v, rows_v, sem):
        wid = lax.axis_index("s") * 2 + lax.axis_index("c")

        @pl.loop(0, wins_per_w)
        def _(w):
            base = (wid * wins_per_w + w) * SC_WIN
            pltpu.sync_copy(idx_hbm.at[pl.ds(base, SC_WIN)], idx_v)
            pltpu.async_copy(table_hbm.at[idx_v], rows_v, sem).wait()
            pltpu.sync_copy(rows_v, out_hbm.at[pl.ds(base, SC_WIN)])

    return _k(table, gpos)


def _rotations():
    rot = jax.random.normal(jax.random.key(42), (1, DIM_HEAD, N_HASHES, N_BUCKETS // 2),
                            dtype=_F32)
    return rot[0].reshape(DIM_HEAD, N_HASHES * (N_BUCKETS // 2))


@jax.jit
def kernel(x, Wqk, Wv, Wout):
    b, t, e = x.shape
    x2 = x.reshape(t, e)
    rot = _rotations()
    t_incl, t_excl, u_strict = _tri_consts()
    dmask = _diag_mask()
    wqk_h = Wqk.reshape(DIM, HEADS, DIM_HEAD).transpose(1, 0, 2)
    wv_h = Wv.reshape(DIM, HEADS, DIM_HEAD).transpose(1, 0, 2)

    HG = HEADS // 2
    group_uns = []
    for g in range(2):
        sl = slice(g * HG, (g + 1) * HG)
        comb, gpos, btid = pl.pallas_call(
            _proj_hash_body,
            grid=(HG,),
            in_specs=[
                pl.BlockSpec((SEQ, DIM), lambda h: (0, 0)),
                pl.BlockSpec((1, DIM, DIM_HEAD), lambda h: (h, 0, 0)),
                pl.BlockSpec((1, DIM, DIM_HEAD), lambda h: (h, 0, 0)),
                pl.BlockSpec((DIM_HEAD, 128), lambda h: (0, 0)),
                pl.BlockSpec((64, 64), lambda h: (0, 0)),
                pl.BlockSpec((64, 64), lambda h: (0, 0)),
                pl.BlockSpec((64, 64), lambda h: (0, 0)),
            ],
            out_specs=[
                pl.BlockSpec((1, SEQ, 128), lambda h: (h, 0, 0)),
                pl.BlockSpec((1, 16, 1024), lambda h: (h, 0, 0)),
                pl.BlockSpec((1, 8, 64), lambda h: (h, 0, 0)),
            ],
            out_shape=[
                jax.ShapeDtypeStruct((HG, SEQ, 128), _F32),
                jax.ShapeDtypeStruct((HG, 16, 1024), jnp.int32),
                jax.ShapeDtypeStruct((HG, 8, 64), _F32),
            ],
        )(x2, wqk_h[sl], wv_h[sl], rot, t_incl, t_excl, u_strict)

        gpos_flat = gpos.reshape(HG * N_HASHES * SEQ)

        so_comb = _sc_scatter_rows(comb.reshape(HG * SEQ, 128), gpos_flat)

        so_view = so_comb.reshape(HG, N_CHUNKS, BUCKET_SIZE, 128)
        so_att = pl.pallas_call(
            _attn_body,
            grid=(HG, 4),
            in_specs=[
                pl.BlockSpec((1, 64, BUCKET_SIZE, 128),
                             lambda h, q: (h, q, 0, 0)),
                pl.BlockSpec((1, 1, BUCKET_SIZE, 128),
                             lambda h, q: (h, (64 * q + N_CHUNKS - 1) % N_CHUNKS, 0, 0)),
                pl.BlockSpec((1, 8, 64), lambda h, q: (h, 0, 0)),
                pl.BlockSpec((64, 128), lambda h, q: (0, 0)),
            ],
            out_specs=pl.BlockSpec((1, 64, BUCKET_SIZE, 128),
                                   lambda h, q: (h, q, 0, 0)),
            out_shape=jax.ShapeDtypeStruct((HG, N_CHUNKS, BUCKET_SIZE, 128), _F32),
        )(so_view, so_view, btid, dmask)

        uns = _sc_gather_rows(so_att.reshape(-1, 128), gpos_flat)
        group_uns.append(uns.reshape(HG, N_HASHES, SEQ, 128))

    TB = 512
    out = pl.pallas_call(
        _combine_body,
        grid=(SEQ // TB,),
        in_specs=[
            pl.BlockSpec((HG, N_HASHES, TB, 128), lambda i: (0, 0, i, 0)),
            pl.BlockSpec((HG, N_HASHES, TB, 128), lambda i: (0, 0, i, 0)),
            pl.BlockSpec((DIM, DIM), lambda i: (0, 0)),
        ],
        out_specs=pl.BlockSpec((TB, DIM), lambda i: (i, 0)),
        out_shape=jax.ShapeDtypeStruct((SEQ, DIM), _F32),
    )(group_uns[0], group_uns[1], Wout)

    return out.reshape(b, t, e)


# speedup vs baseline: 1.1841x; 1.1841x over previous
import functools

import jax
import jax.numpy as jnp
import numpy as np
from jax import lax
from jax.experimental import pallas as pl
from jax.experimental.pallas import tpu as pltpu
from jax.experimental.pallas import tpu_sc as plsc

DIM = 768
HEADS = 12
DIM_HEAD = 64
SEQ = 4096
BUCKET_SIZE = 64
N_HASHES = 4
N_BUCKETS = SEQ // BUCKET_SIZE
N_CHUNKS = N_HASHES * N_BUCKETS
TOKEN_SELF_ATTN_VALUE = -5e4

_F32 = jnp.float32


def _tri_consts():
    i = np.arange(64)
    t_incl = (i[None, :] <= i[:, None]).astype(np.float32)
    t_excl = (i[None, :] < i[:, None]).astype(np.float32)
    u_strict = (i[:, None] < i[None, :]).astype(np.float32)
    return jnp.asarray(t_incl), jnp.asarray(t_excl), jnp.asarray(u_strict)


def _diag_mask():
    m = np.zeros((64, 128), np.float32)
    m[np.arange(64), np.arange(64)] = 1.0
    return jnp.asarray(m)


def _proj_hash_body(x_ref, wqk_ref, wv_ref, rot_ref, ti_ref, te_ref, sel_ref,
                    selt_ref, us_ref, comb_ref, gpos_ref, btid_ref):
    h = pl.program_id(0)
    x = x_ref[...]
    qk = jnp.dot(x, wqk_ref[0], preferred_element_type=_F32)
    v = jnp.dot(x, wv_ref[0], preferred_element_type=_F32)
    comb_ref[0, :, 0:DIM_HEAD] = qk
    comb_ref[0, :, DIM_HEAD:2 * DIM_HEAD] = v

    p = jnp.dot(qk, rot_ref[...], preferred_element_type=_F32)
    t_incl, t_excl, u_strict = ti_ref[...], te_ref[...], us_ref[...]
    sel, selt = sel_ref[...], selt_ref[...]
    lane = jax.lax.broadcasted_iota(jnp.int32, (SEQ, N_BUCKETS), 1)
    tvec_row = jax.lax.broadcasted_iota(jnp.int32, (1, SEQ), 1).astype(_F32)
    for r in range(N_HASHES):
        scores = p[:, r * 64:(r + 1) * 64]
        m = jnp.max(scores, axis=1, keepdims=True)
        b_idx = jnp.min(jnp.where(scores == m, lane, N_BUCKETS),
                        axis=1, keepdims=True)
        onehot = (lane == b_idx).astype(_F32)
        o3 = onehot.reshape(64, 64, N_BUCKETS)
        inner = jnp.einsum('ij,bjk->bik', t_incl, o3,
                           preferred_element_type=_F32)
        c_in = inner.reshape(SEQ, N_BUCKETS)
        bsum = jnp.dot(sel, onehot, preferred_element_type=_F32)
        blk_offs = jnp.dot(t_excl, bsum,
                           preferred_element_type=_F32)
        bo2d = jnp.dot(selt, blk_offs, preferred_element_type=_F32)
        counts = jnp.sum(bsum, axis=0, keepdims=True)
        b_offs = jnp.dot(counts, u_strict, preferred_element_type=_F32)
        pmat = onehot * (c_in - 1.0 + bo2d + b_offs)
        pos = jnp.sum(pmat.reshape(64, 64, N_BUCKETS), axis=2)
        base = (h * N_HASHES + r) * SEQ
        gpos = pos.astype(jnp.int32) + base
        gpos_ref[0, r * 4:(r + 1) * 4, :] = gpos.reshape(4, 1024)

        pos_flat = pos.reshape(SEQ, 1)
        slot_lane = jax.lax.broadcasted_iota(jnp.int32, (SEQ, 128), 1)
        slots = jnp.where(slot_lane < 64, slot_lane, slot_lane + (SEQ - 128))
        hit = (pos_flat.astype(jnp.int32) == slots).astype(_F32)
        btid = jnp.dot(tvec_row, hit, preferred_element_type=_F32)
        btid_ref[0, 2 * r:2 * r + 1, :] = btid[:, 0:64]
        btid_ref[0, 2 * r + 1:2 * r + 2, :] = btid[:, 64:128]


def _attn_body(cur_ref, halo_ref, btid_ref, dmask_ref, out_ref):
    q = pl.program_id(1)
    cur = cur_ref[0]
    halo = halo_ref[0, 0]
    prevs = jnp.concatenate([halo[None], cur[:-1]], axis=0)
    kv = jnp.concatenate([cur, prevs], axis=1)
    bq = cur[:, :, 0:DIM_HEAD]
    bqk_k = kv[:, :, 0:DIM_HEAD]
    norm = jnp.sqrt(jnp.sum(bqk_k * bqk_k, axis=2, keepdims=True))
    bk = bqk_k / jnp.maximum(norm, 1e-12)
    bv = kv[:, :, DIM_HEAD:2 * DIM_HEAD]
    dots = jax.lax.dot_general(
        bq, bk, (((2,), (2,)), ((0,), (0,))),
        preferred_element_type=_F32) * (DIM_HEAD ** -0.5)
    dm = dmask_ref[...][None]
    dots = dots - dm * dots + TOKEN_SELF_ATTN_VALUE * dm
    qt_b = btid_ref[0, pl.ds(2 * q, 1), :]
    r_prev = (q + N_HASHES - 1) % N_HASHES
    kt_b = btid_ref[0, pl.ds(2 * r_prev + 1, 1), :]
    ktx = jnp.concatenate(
        [jnp.full((1, 64), -1.0, _F32), kt_b], axis=1)
    qt_col = jnp.broadcast_to(qt_b, (64, 64)).T[:, 0:1]
    bnd = (qt_col[None] == ktx[:, None, :]).astype(_F32)
    d0 = dots[0:1]
    d0 = d0 - bnd * d0 + TOKEN_SELF_ATTN_VALUE * bnd
    dots = jnp.concatenate([d0, dots[1:]], axis=0)
    m = jnp.max(dots, axis=2, keepdims=True)
    ex = jnp.exp(dots - m)
    s = jnp.sum(ex, axis=2, keepdims=True)
    lse = m + jnp.log(s)
    pr = ex / s
    bo = jax.lax.dot_general(
        pr, bv, (((2,), (1,)), ((0,), (0,))),
        preferred_element_type=_F32)
    out_ref[0, :, :, 0:DIM_HEAD] = bo
    out_ref[0, :, :, DIM_HEAD:2 * DIM_HEAD] = jnp.broadcast_to(
        lse, (64, 64, DIM_HEAD))


def _combine_body(u0_ref, u1_ref, wout_ref, out_ref):
    ys = []
    for ref in (u0_ref, u1_ref):
        u = ref[...]
        logit = u[:, :, :, DIM_HEAD:DIM_HEAD + 1]
        m = jnp.max(logit, axis=1, keepdims=True)
        ex = jnp.exp(logit - m)
        s = jnp.sum(ex, axis=1, keepdims=True)
        lse = m + jnp.log(s)
        probs = jnp.exp(logit - lse)
        y_h = jnp.sum(u[:, :, :, 0:DIM_HEAD] * probs, axis=1)
        ys.extend([y_h[i] for i in range(y_h.shape[0])])
    y = jnp.concatenate(ys, axis=1)
    out_ref[...] = jnp.dot(y, wout_ref[...], preferred_element_type=_F32)


SC_WIN = 128
SC_NW = 32

def _sc_mesh():
    return plsc.VectorSubcoreMesh(core_axis_name="c", subcore_axis_name="s")


def _sc_scatter_rows(comb2, gpos):
    d = comb2.shape[1]
    n = gpos.shape[0]
    wins_per_w = n // (SC_WIN * SC_NW)

    @functools.partial(
        pl.kernel, mesh=_sc_mesh(),
        out_type=jax.ShapeDtypeStruct((n, d), comb2.dtype),
        scratch_types=[
            pltpu.VMEM((SC_WIN,), jnp.int32),
            pltpu.VMEM((SC_WIN, d), comb2.dtype),
        ],
    )
    def _k(comb_hbm, idx_hbm, out_hbm, idx_v, rows_v):
        wid = lax.axis_index("s") * 2 + lax.axis_index("c")

        @pl.loop(0, wins_per_w)
        def _(w):
            base = (wid * wins_per_w + w) * SC_WIN
            src = (base // (N_HASHES * SEQ)) * SEQ + base % SEQ
            pltpu.sync_copy(idx_hbm.at[pl.ds(base, SC_WIN)], idx_v)
            pltpu.sync_copy(comb_hbm.at[pl.ds(src, SC_WIN)], rows_v)
            pltpu.sync_copy(rows_v, out_hbm.at[idx_v])

    return _k(comb2, gpos)


def _sc_gather_rows(table, gpos):
    d = table.shape[1]
    n = gpos.shape[0]
    wins_per_w = n // (SC_WIN * SC_NW)

    @functools.partial(
        pl.kernel, mesh=_sc_mesh(),
        out_type=jax.ShapeDtypeStruct((n, d), table.dtype),
        scratch_types=[
            pltpu.VMEM((SC_WIN,), jnp.int32),
            pltpu.VMEM((SC_WIN, d), table.dtype),
            pltpu.SemaphoreType.DMA,
        ],
    )
    def _k(table_hbm, idx_hbm, out_hbm, idx_v, rows_v, sem):
        wid = lax.axis_index("s") * 2 + lax.axis_index("c")

        @pl.loop(0, wins_per_w)
        def _(w):
            base = (wid * wins_per_w + w) * SC_WIN
            pltpu.sync_copy(idx_hbm.at[pl.ds(base, SC_WIN)], idx_v)
            pltpu.async_copy(table_hbm.at[idx_v], rows_v, sem).wait()
            pltpu.sync_copy(rows_v, out_hbm.at[pl.ds(base, SC_WIN)])

    return _k(table, gpos)


def _rotations():
    rot = jax.random.normal(jax.random.key(42), (1, DIM_HEAD, N_HASHES, N_BUCKETS // 2),
                            dtype=_F32)[0]
    cols = []
    for r in range(N_HASHES):
        cols.append(rot[:, r, :])
        cols.append(-rot[:, r, :])
    return jnp.concatenate(cols, axis=1)


def _sel_consts():
    blk = np.repeat(np.arange(64), 64)
    sel = (np.arange(64)[:, None] == blk[None, :]).astype(np.float32)
    return jnp.asarray(sel), jnp.asarray(sel.T)


@jax.jit
def kernel(x, Wqk, Wv, Wout):
    b, t, e = x.shape
    x2 = x.reshape(t, e)
    rot = _rotations()
    t_incl, t_excl, u_strict = _tri_consts()
    sel, selt = _sel_consts()
    dmask = _diag_mask()
    wqk_h = Wqk.reshape(DIM, HEADS, DIM_HEAD).transpose(1, 0, 2)
    wv_h = Wv.reshape(DIM, HEADS, DIM_HEAD).transpose(1, 0, 2)

    HG = HEADS // 2
    group_uns = []
    for g in range(2):
        sl = slice(g * HG, (g + 1) * HG)
        comb, gpos, btid = pl.pallas_call(
            _proj_hash_body,
            grid=(HG,),
            in_specs=[
                pl.BlockSpec((SEQ, DIM), lambda h: (0, 0)),
                pl.BlockSpec((1, DIM, DIM_HEAD), lambda h: (h, 0, 0)),
                pl.BlockSpec((1, DIM, DIM_HEAD), lambda h: (h, 0, 0)),
                pl.BlockSpec((DIM_HEAD, 256), lambda h: (0, 0)),
                pl.BlockSpec((64, 64), lambda h: (0, 0)),
                pl.BlockSpec((64, 64), lambda h: (0, 0)),
                pl.BlockSpec((64, SEQ), lambda h: (0, 0)),
                pl.BlockSpec((SEQ, 64), lambda h: (0, 0)),
                pl.BlockSpec((64, 64), lambda h: (0, 0)),
            ],
            out_specs=[
                pl.BlockSpec((1, SEQ, 128), lambda h: (h, 0, 0)),
                pl.BlockSpec((1, 16, 1024), lambda h: (h, 0, 0)),
                pl.BlockSpec((1, 8, 64), lambda h: (h, 0, 0)),
            ],
            out_shape=[
                jax.ShapeDtypeStruct((HG, SEQ, 128), _F32),
                jax.ShapeDtypeStruct((HG, 16, 1024), jnp.int32),
                jax.ShapeDtypeStruct((HG, 8, 64), _F32),
            ],
        )(x2, wqk_h[sl], wv_h[sl], rot, t_incl, t_excl, sel, selt, u_strict)

        gpos_flat = gpos.reshape(HG * N_HASHES * SEQ)

        so_comb = _sc_scatter_rows(comb.reshape(HG * SEQ, 128), gpos_flat)

        so_view = so_comb.reshape(HG, N_CHUNKS, BUCKET_SIZE, 128)
        so_att = pl.pallas_call(
            _attn_body,
            grid=(HG, 4),
            in_specs=[
                pl.BlockSpec((1, 64, BUCKET_SIZE, 128),
                             lambda h, q: (h, q, 0, 0)),
                pl.BlockSpec((1, 1, BUCKET_SIZE, 128),
                             lambda h, q: (h, (64 * q + N_CHUNKS - 1) % N_CHUNKS, 0, 0)),
                pl.BlockSpec((1, 8, 64), lambda h, q: (h, 0, 0)),
                pl.BlockSpec((64, 128), lambda h, q: (0, 0)),
            ],
            out_specs=pl.BlockSpec((1, 64, BUCKET_SIZE, 128),
                                   lambda h, q: (h, q, 0, 0)),
            out_shape=jax.ShapeDtypeStruct((HG, N_CHUNKS, BUCKET_SIZE, 128), _F32),
        )(so_view, so_view, btid, dmask)

        uns = _sc_gather_rows(so_att.reshape(-1, 128), gpos_flat)
        group_uns.append(uns.reshape(HG, N_HASHES, SEQ, 128))

    TB = 512
    out = pl.pallas_call(
        _combine_body,
        grid=(SEQ // TB,),
        in_specs=[
            pl.BlockSpec((HG, N_HASHES, TB, 128), lambda i: (0, 0, i, 0)),
            pl.BlockSpec((HG, N_HASHES, TB, 128), lambda i: (0, 0, i, 0)),
            pl.BlockSpec((DIM, DIM), lambda i: (0, 0)),
        ],
        out_specs=pl.BlockSpec((TB, DIM), lambda i: (i, 0)),
        out_shape=jax.ShapeDtypeStruct((SEQ, DIM), _F32),
    )(group_uns[0], group_uns[1], Wout)

    return out.reshape(b, t, e)
